# Initial kernel scaffold; baseline (speedup 1.0000x reference)
import jax
import jax.numpy as jnp
from jax.experimental import pallas as pl


def kernel(node_feats, edge_index, edge_attr, W, b):
    raise NotImplementedError("write your pallas kernel here")


# speedup vs baseline: 3.8041x; 3.8041x over previous
import functools

import jax
import jax.numpy as jnp
from jax import lax
from jax.experimental import pallas as pl
from jax.experimental.pallas import tpu as pltpu
from jax.experimental.pallas import tpu_sc as plsc

N_NODES = 10000
N_EDGES = 320000
D_FEAT = 128
D_EDGE = 16
D_OUT = 128

NUM_CORES = 2
NUM_SUBCORES = 16
NUM_WORKERS = NUM_CORES * NUM_SUBCORES
E_PER_W = N_EDGES // NUM_WORKERS
CHUNK = 400
N_CHUNKS = E_PER_W // CHUNK
SUB = 80
N_SUB = CHUNK // SUB

R_BLK = 4000


def _pq_body(nf_ref, w1_ref, w2_ref, p_ref, q_ref):
    nf = nf_ref[...]
    p_ref[...] = jnp.dot(nf, w1_ref[...], preferred_element_type=jnp.float32)
    q_ref[...] = jnp.dot(nf, w2_ref[...], preferred_element_type=jnp.float32)


def _r_body(ea_ref, w3_ref, b_ref, r_ref):
    r_ref[...] = (
        jnp.dot(ea_ref[...], w3_ref[...], preferred_element_type=jnp.float32)
        + b_ref[...]
    )


def _sc_combine(p_hbm, q_hbm, r_hbm, row_hbm, col_hbm, out_hbm,
                row_v, col_v, buf, sem_g):
    wid = lax.axis_index("s") * NUM_CORES + lax.axis_index("c")
    base = wid * E_PER_W
    pltpu.sync_copy(row_hbm.at[pl.ds(base, E_PER_W)], row_v)
    pltpu.sync_copy(col_hbm.at[pl.ds(base, E_PER_W)], col_v)

    def chunk_body(j, carry):
        off = j * CHUNK
        pltpu.sync_copy(r_hbm.at[pl.ds(base + off, CHUNK)], buf)
        copies = []
        for k in range(N_SUB):
            idx_off = off + k * SUB
            dst = buf.at[pl.ds(k * SUB, SUB)]
            copies.append(pltpu.async_copy(
                p_hbm.at[row_v.at[pl.ds(idx_off, SUB)]], dst, sem_g,
                add=True))
            copies.append(pltpu.async_copy(
                q_hbm.at[col_v.at[pl.ds(idx_off, SUB)]], dst, sem_g,
                add=True))
        for cp in copies:
            cp.wait()
        pltpu.sync_copy(buf, out_hbm.at[pl.ds(base + off, CHUNK)])
        return carry

    lax.fori_loop(0, N_CHUNKS, chunk_body, 0)


def kernel(node_feats, edge_index, edge_attr, W, b):
    row = edge_index[0].astype(jnp.int32)
    col = edge_index[1].astype(jnp.int32)
    w1 = W[:D_FEAT]
    w2 = W[D_FEAT:2 * D_FEAT]
    w3 = W[2 * D_FEAT:]
    b2 = b.reshape(1, D_OUT)

    p, q = pl.pallas_call(
        _pq_body,
        out_shape=(
            jax.ShapeDtypeStruct((N_NODES, D_FEAT), jnp.float32),
            jax.ShapeDtypeStruct((N_NODES, D_FEAT), jnp.float32),
        ),
    )(node_feats, w1, w2)

    r = pl.pallas_call(
        _r_body,
        grid=(N_EDGES // R_BLK,),
        in_specs=[
            pl.BlockSpec((R_BLK, D_EDGE), lambda i: (i, 0)),
            pl.BlockSpec((D_EDGE, D_OUT), lambda i: (0, 0)),
            pl.BlockSpec((1, D_OUT), lambda i: (0, 0)),
        ],
        out_specs=pl.BlockSpec((R_BLK, D_OUT), lambda i: (i, 0)),
        out_shape=jax.ShapeDtypeStruct((N_EDGES, D_OUT), jnp.float32),
    )(edge_attr, w3, b2)

    mesh = plsc.VectorSubcoreMesh(
        core_axis_name="c", subcore_axis_name="s",
        num_cores=NUM_CORES, num_subcores=NUM_SUBCORES)
    combine = functools.partial(
        pl.kernel,
        out_type=jax.ShapeDtypeStruct((N_EDGES, D_OUT), jnp.float32),
        mesh=mesh,
        scratch_types=[
            pltpu.VMEM((E_PER_W,), jnp.int32),
            pltpu.VMEM((E_PER_W,), jnp.int32),
            pltpu.VMEM((CHUNK, D_OUT), jnp.float32),
            pltpu.SemaphoreType.DMA,
        ],
    )(_sc_combine)

    return combine(p, q, r, row, col)


# speedup vs baseline: 5.2640x; 1.3838x over previous
import functools

import jax
import jax.numpy as jnp
from jax import lax
from jax.experimental import pallas as pl
from jax.experimental.pallas import tpu as pltpu
from jax.experimental.pallas import tpu_sc as plsc

N_NODES = 10000
N_EDGES = 320000
D_FEAT = 128
D_EDGE = 16
D_OUT = 128

NUM_CORES = 2
NUM_SUBCORES = 16
NUM_WORKERS = NUM_CORES * NUM_SUBCORES
E_PER_W = N_EDGES // NUM_WORKERS
CHUNK = 400
N_CHUNKS = E_PER_W // CHUNK
SUB = 80
N_SUB = CHUNK // SUB

R_BLK = 16000


def _pq_body(nf_ref, w1_ref, w2_ref, p_ref, q_ref):
    nf = nf_ref[...]
    p_ref[...] = jnp.dot(nf, w1_ref[...], preferred_element_type=jnp.float32)
    q_ref[...] = jnp.dot(nf, w2_ref[...], preferred_element_type=jnp.float32)


def _r_body(ea_t_ref, w3_ref, b_ref, r_ref):
    r_ref[...] = (
        jax.lax.dot_general(
            ea_t_ref[...], w3_ref[...],
            dimension_numbers=(((0,), (0,)), ((), ())),
            preferred_element_type=jnp.float32)
        + b_ref[...]
    )


def _sc_combine(p_hbm, q_hbm, r_hbm, row_hbm, col_hbm, out_hbm,
                row_v, col_v, buf, sem_g):
    wid = lax.axis_index("s") * NUM_CORES + lax.axis_index("c")
    base = wid * E_PER_W
    pltpu.sync_copy(row_hbm.at[pl.ds(base, E_PER_W)], row_v)
    pltpu.sync_copy(col_hbm.at[pl.ds(base, E_PER_W)], col_v)

    def chunk_body(j, carry):
        off = j * CHUNK
        pltpu.sync_copy(r_hbm.at[pl.ds(base + off, CHUNK)], buf)
        copies = []
        for k in range(N_SUB):
            idx_off = off + k * SUB
            dst = buf.at[pl.ds(k * SUB, SUB)]
            copies.append(pltpu.async_copy(
                p_hbm.at[row_v.at[pl.ds(idx_off, SUB)]], dst, sem_g,
                add=True))
            copies.append(pltpu.async_copy(
                q_hbm.at[col_v.at[pl.ds(idx_off, SUB)]], dst, sem_g,
                add=True))
        for cp in copies:
            cp.wait()
        pltpu.sync_copy(buf, out_hbm.at[pl.ds(base + off, CHUNK)])
        return carry

    lax.fori_loop(0, N_CHUNKS, chunk_body, 0)


def kernel(node_feats, edge_index, edge_attr, W, b):
    row = edge_index[0].astype(jnp.int32)
    col = edge_index[1].astype(jnp.int32)
    w1 = W[:D_FEAT]
    w2 = W[D_FEAT:2 * D_FEAT]
    w3 = W[2 * D_FEAT:]
    b2 = b.reshape(1, D_OUT)

    p, q = pl.pallas_call(
        _pq_body,
        out_shape=(
            jax.ShapeDtypeStruct((N_NODES, D_FEAT), jnp.float32),
            jax.ShapeDtypeStruct((N_NODES, D_FEAT), jnp.float32),
        ),
    )(node_feats, w1, w2)

    r = pl.pallas_call(
        _r_body,
        grid=(N_EDGES // R_BLK,),
        in_specs=[
            pl.BlockSpec((D_EDGE, R_BLK), lambda i: (0, i)),
            pl.BlockSpec((D_EDGE, D_OUT), lambda i: (0, 0)),
            pl.BlockSpec((1, D_OUT), lambda i: (0, 0)),
        ],
        out_specs=pl.BlockSpec((R_BLK, D_OUT), lambda i: (i, 0)),
        out_shape=jax.ShapeDtypeStruct((N_EDGES, D_OUT), jnp.float32),
    )(edge_attr.T, w3, b2)

    mesh = plsc.VectorSubcoreMesh(
        core_axis_name="c", subcore_axis_name="s",
        num_cores=NUM_CORES, num_subcores=NUM_SUBCORES)
    combine = functools.partial(
        pl.kernel,
        out_type=jax.ShapeDtypeStruct((N_EDGES, D_OUT), jnp.float32),
        mesh=mesh,
        scratch_types=[
            pltpu.VMEM((E_PER_W,), jnp.int32),
            pltpu.VMEM((E_PER_W,), jnp.int32),
            pltpu.VMEM((CHUNK, D_OUT), jnp.float32),
            pltpu.SemaphoreType.DMA,
        ],
    )(_sc_combine)

    return combine(p, q, r, row, col)


# speedup vs baseline: 5.4414x; 1.0337x over previous
import functools

import jax
import jax.numpy as jnp
from jax import lax
from jax.experimental import pallas as pl
from jax.experimental.pallas import tpu as pltpu
from jax.experimental.pallas import tpu_sc as plsc

N_NODES = 10000
N_EDGES = 320000
D_FEAT = 128
D_EDGE = 16
D_OUT = 128

NUM_CORES = 2
NUM_SUBCORES = 16
NUM_WORKERS = NUM_CORES * NUM_SUBCORES
E_PER_W = N_EDGES // NUM_WORKERS
CHUNK = 400
N_CHUNKS = E_PER_W // CHUNK
SUB = 80
N_SUB = CHUNK // SUB

F_BLK = 16000


def _pq_body(nf_ref, w1_ref, w2_ref, p_ref, q_ref):
    nf = nf_ref[...]
    p_ref[...] = jnp.dot(nf, w1_ref[...], preferred_element_type=jnp.float32)
    q_ref[...] = jnp.dot(nf, w2_ref[...], preferred_element_type=jnp.float32)


def _fin_body(s_ref, ea_t_ref, w3_ref, b_ref, o_ref):
    o_ref[...] = (
        s_ref[...]
        + jax.lax.dot_general(
            ea_t_ref[...], w3_ref[...],
            dimension_numbers=(((0,), (0,)), ((), ())),
            preferred_element_type=jnp.float32)
        + b_ref[...]
    )


def _sc_gather(p_hbm, q_hbm, ei_hbm, s_hbm, row_v, col_v, buf, sem_g):
    wid = lax.axis_index("s") * NUM_CORES + lax.axis_index("c")
    base = wid * E_PER_W
    pltpu.sync_copy(ei_hbm.at[pl.ds(base, E_PER_W)], row_v)
    pltpu.sync_copy(ei_hbm.at[pl.ds(N_EDGES + base, E_PER_W)], col_v)

    def chunk_body(j, carry):
        off = j * CHUNK
        copies = []
        for k in range(N_SUB):
            idx_off = off + k * SUB
            dst = buf.at[pl.ds(k * SUB, SUB)]
            copies.append(pltpu.async_copy(
                p_hbm.at[row_v.at[pl.ds(idx_off, SUB)]], dst, sem_g))
        for cp in copies:
            cp.wait()
        copies = []
        for k in range(N_SUB):
            idx_off = off + k * SUB
            dst = buf.at[pl.ds(k * SUB, SUB)]
            copies.append(pltpu.async_copy(
                q_hbm.at[col_v.at[pl.ds(idx_off, SUB)]], dst, sem_g,
                add=True))
        for cp in copies:
            cp.wait()
        pltpu.sync_copy(buf, s_hbm.at[pl.ds(base + off, CHUNK)])
        return carry

    lax.fori_loop(0, N_CHUNKS, chunk_body, 0)


def kernel(node_feats, edge_index, edge_attr, W, b):
    ei = edge_index.astype(jnp.int32).reshape(-1)
    w1 = W[:D_FEAT]
    w2 = W[D_FEAT:2 * D_FEAT]
    w3 = W[2 * D_FEAT:]
    b2 = b.reshape(1, D_OUT)

    p, q = pl.pallas_call(
        _pq_body,
        out_shape=(
            jax.ShapeDtypeStruct((N_NODES, D_FEAT), jnp.float32),
            jax.ShapeDtypeStruct((N_NODES, D_FEAT), jnp.float32),
        ),
    )(node_feats, w1, w2)

    mesh = plsc.VectorSubcoreMesh(
        core_axis_name="c", subcore_axis_name="s",
        num_cores=NUM_CORES, num_subcores=NUM_SUBCORES)
    gather = functools.partial(
        pl.kernel,
        out_type=jax.ShapeDtypeStruct((N_EDGES, D_OUT), jnp.float32),
        mesh=mesh,
        scratch_types=[
            pltpu.VMEM((E_PER_W,), jnp.int32),
            pltpu.VMEM((E_PER_W,), jnp.int32),
            pltpu.VMEM((CHUNK, D_OUT), jnp.float32),
            pltpu.SemaphoreType.DMA,
        ],
    )(_sc_gather)
    s = gather(p, q, ei)

    return pl.pallas_call(
        _fin_body,
        grid=(N_EDGES // F_BLK,),
        in_specs=[
            pl.BlockSpec((F_BLK, D_OUT), lambda i: (i, 0)),
            pl.BlockSpec((D_EDGE, F_BLK), lambda i: (0, i)),
            pl.BlockSpec((D_EDGE, D_OUT), lambda i: (0, 0)),
            pl.BlockSpec((1, D_OUT), lambda i: (0, 0)),
        ],
        out_specs=pl.BlockSpec((F_BLK, D_OUT), lambda i: (i, 0)),
        out_shape=jax.ShapeDtypeStruct((N_EDGES, D_OUT), jnp.float32),
    )(s, edge_attr.T, w3, b2)
